# baseline (speedup 1.0000x reference)
import functools

import jax
import jax.numpy as jnp
from jax import lax
from jax.experimental import pallas as pl
from jax.experimental.pallas import tpu as pltpu
from jax.experimental.pallas import tpu_sc as plsc

PATCH = 16
K = 1024
NUM_CLASSES = 10
CPAD = 16
ROWS_BLK = 128
NC, NS = 2, 16


def _tok_body(x_ref, c_ref, o_ref):
    x = x_ref[...]
    c = c_ref[...]
    xc = lax.dot_general(c, x, (((1,), (1,)), ((), ())),
                         preferred_element_type=jnp.float32)
    c2 = jnp.sum(c * c, axis=1)
    dist = c2[:, None] - 2.0 * xc
    m = jnp.min(dist, axis=0, keepdims=True)
    rows = lax.broadcasted_iota(jnp.int32, dist.shape, 0)
    tok = jnp.min(jnp.where(dist == m, rows, K), axis=0)
    o_ref[...] = tok.reshape(1, 1, ROWS_BLK)


def _tc_tokens(patches, centers):
    n = patches.shape[0]
    grid = n // ROWS_BLK
    out = pl.pallas_call(
        _tok_body,
        grid=(grid,),
        in_specs=[
            pl.BlockSpec((ROWS_BLK, patches.shape[1]), lambda i: (i, 0)),
            pl.BlockSpec(centers.shape, lambda i: (0, 0)),
        ],
        out_specs=pl.BlockSpec((1, 1, ROWS_BLK), lambda i: (i, 0, 0)),
        out_shape=jax.ShapeDtypeStruct((grid, 1, ROWS_BLK), jnp.int32),
    )(patches, centers)
    return out.reshape(n)


_PK = 196 * K
_VALID = 196


def _sc_body(idx_hbm, table_hbm, out_hbm, idx_v, idxc_v, rows_v, acc_v, sem):
    cid = lax.axis_index("c")
    sid = lax.axis_index("s")
    wid = sid * NC + cid
    n_b = idx_hbm.shape[0]
    b_per_w = n_b // (NC * NS)
    lane = lax.iota(jnp.int32, CPAD)

    def build_and_fire(i):
        b = wid * b_per_w + i
        pltpu.sync_copy(idx_hbm.at[b], idx_v.at[i])
        for c in range(NUM_CLASSES):
            off = jnp.full((CPAD,), c * _PK, jnp.int32)
            for h in range(2):
                for j in range(8):
                    idxc_v[i, pl.ds((2 * c + h) * 128 + 16 * j, 16)] = (
                        idx_v[i, h, pl.ds(16 * j, 16)] + off)
        return pltpu.async_copy(table_hbm.at[idxc_v.at[i]], rows_v.at[i], sem)

    def reduce_and_store(i, cp):
        b = wid * b_per_w + i
        cp.wait()
        out_acc = jnp.zeros((CPAD,), jnp.float32)
        for c in range(NUM_CLASSES):
            a0 = jnp.zeros((CPAD,), jnp.float32)
            a1 = jnp.zeros((CPAD,), jnp.float32)
            for j in range(8):
                a0 = a0 + rows_v[i, pl.ds(2 * c * 128 + 16 * j, 16)]
            for j in range(4):
                a1 = a1 + rows_v[i, pl.ds((2 * c + 1) * 128 + 16 * j, 16)]
            tail = rows_v[i, pl.ds((2 * c + 1) * 128 + 64, 16)]
            a1 = a1 + jnp.where(lane < _VALID - 192, tail, 0.0)
            s = jnp.sum(a0 + a1)
            out_acc = out_acc + jnp.where(lane == c, s, 0.0)
        acc_v[...] = out_acc
        pltpu.sync_copy(acc_v, out_hbm.at[b])

    cps = [build_and_fire(i) for i in range(b_per_w)]
    for i in range(b_per_w):
        reduce_and_store(i, cps[i])


def _sc_logits(idx, table_flat):
    n_b = idx.shape[0]
    b_per_w = n_b // (NC * NS)
    mesh = plsc.VectorSubcoreMesh(core_axis_name="c", subcore_axis_name="s")
    f = functools.partial(
        pl.kernel,
        mesh=mesh,
        out_type=jax.ShapeDtypeStruct((n_b, CPAD), jnp.float32),
        scratch_types=[
            pltpu.VMEM((b_per_w, 2, 128), jnp.int32),
            pltpu.VMEM((b_per_w, 2 * NUM_CLASSES * 128), jnp.int32),
            pltpu.VMEM((b_per_w, 2 * NUM_CLASSES * 128), jnp.float32),
            pltpu.VMEM((CPAD,), jnp.float32),
            pltpu.SemaphoreType.DMA,
        ],
        compiler_params=pltpu.CompilerParams(use_tc_tiling_on_sc=False,
                                             needs_layout_passes=False),
    )(_sc_body)
    return f(idx, table_flat)


def kernel(inputs, centers, class_log_probs):
    b, h, w = inputs.shape
    p = PATCH
    gh, gw = h // p, w // p
    pp = gh * gw
    patches = (inputs.reshape(b, gh, p, gw, p)
               .transpose(0, 1, 3, 2, 4)
               .reshape(b * pp, p * p))
    tok = _tc_tokens(patches, centers)

    idx = tok.reshape(b, pp) + (jnp.arange(pp, dtype=jnp.int32) * K)[None, :]
    idx = jnp.pad(idx, ((0, 0), (0, 256 - pp))).reshape(b, 2, 128)

    out = _sc_logits(idx, class_log_probs.reshape(-1))
    return out[:, :NUM_CLASSES]


# speedup vs baseline: 1.0029x; 1.0029x over previous
import functools

import jax
import jax.numpy as jnp
from jax import lax
from jax.experimental import pallas as pl
from jax.experimental.pallas import tpu as pltpu
from jax.experimental.pallas import tpu_sc as plsc

PATCH = 16
K = 1024
NUM_CLASSES = 10
CPAD = 16
ROWS_BLK = 128
NC, NS = 2, 16


def _tok_body(x_ref, c_ref, o_ref):
    x = x_ref[...]
    c = c_ref[...]
    xc = lax.dot_general(c, x, (((1,), (1,)), ((), ())),
                         preferred_element_type=jnp.float32)
    c2 = jnp.sum(c * c, axis=1)
    dist = c2[:, None] - 2.0 * xc
    m = jnp.min(dist, axis=0, keepdims=True)
    rows = lax.broadcasted_iota(jnp.int32, dist.shape, 0)
    tok = jnp.min(jnp.where(dist == m, rows, K), axis=0)
    g = pl.program_id(0) * ROWS_BLK + lax.broadcasted_iota(
        jnp.int32, (ROWS_BLK,), 0)
    idx = tok + (g % 196) * K
    o_ref[...] = idx.reshape(1, 1, ROWS_BLK)


def _flat_body(x_ref, o_ref):
    o_ref[...] = x_ref[...].reshape(o_ref.shape)


def _tc_flatten(clp):
    c, p, k = clp.shape
    return pl.pallas_call(
        _flat_body,
        grid=(c,),
        in_specs=[pl.BlockSpec((1, p, k), lambda i: (i, 0, 0))],
        out_specs=pl.BlockSpec((p * k,), lambda i: (i,)),
        out_shape=jax.ShapeDtypeStruct((c * p * k,), jnp.float32),
    )(clp)


def _tc_tokens(patches, centers):
    n = patches.shape[0]
    grid = n // ROWS_BLK
    out = pl.pallas_call(
        _tok_body,
        grid=(grid,),
        in_specs=[
            pl.BlockSpec((ROWS_BLK, patches.shape[1]), lambda i: (i, 0)),
            pl.BlockSpec(centers.shape, lambda i: (0, 0)),
        ],
        out_specs=pl.BlockSpec((1, 1, ROWS_BLK), lambda i: (i, 0, 0)),
        out_shape=jax.ShapeDtypeStruct((grid, 1, ROWS_BLK), jnp.int32),
    )(patches, centers)
    return out.reshape(n)


_PK = 196 * K
_VALID = 196


def _sc_body(idx_hbm, table_hbm, out_hbm, idx_v, idxc_v, rows_v, acc_v, sem):
    cid = lax.axis_index("c")
    sid = lax.axis_index("s")
    wid = sid * NC + cid
    n_b = idx_hbm.shape[0]
    b_per_w = n_b // (NC * NS)
    lane = lax.iota(jnp.int32, CPAD)

    def build_and_fire(i):
        b = wid * b_per_w + i
        pltpu.sync_copy(idx_hbm.at[b], idx_v.at[i])
        for c in range(NUM_CLASSES):
            off = jnp.full((CPAD,), c * _PK, jnp.int32)
            for h in range(2):
                for j in range(8):
                    idxc_v[i, pl.ds((2 * c + h) * 128 + 16 * j, 16)] = (
                        idx_v[i, h, pl.ds(16 * j, 16)] + off)
        return pltpu.async_copy(table_hbm.at[idxc_v.at[i]], rows_v.at[i], sem)

    def reduce_and_store(i, cp):
        b = wid * b_per_w + i
        cp.wait()
        out_acc = jnp.zeros((CPAD,), jnp.float32)
        for c in range(NUM_CLASSES):
            a0 = jnp.zeros((CPAD,), jnp.float32)
            a1 = jnp.zeros((CPAD,), jnp.float32)
            for j in range(8):
                a0 = a0 + rows_v[i, pl.ds(2 * c * 128 + 16 * j, 16)]
            for j in range(4):
                a1 = a1 + rows_v[i, pl.ds((2 * c + 1) * 128 + 16 * j, 16)]
            tail = rows_v[i, pl.ds((2 * c + 1) * 128 + 64, 16)]
            a1 = a1 + jnp.where(lane < _VALID - 192, tail, 0.0)
            s = jnp.sum(a0 + a1)
            out_acc = out_acc + jnp.where(lane == c, s, 0.0)
        acc_v[...] = out_acc
        pltpu.sync_copy(acc_v, out_hbm.at[b])

    cps = [build_and_fire(i) for i in range(b_per_w)]
    for i in range(b_per_w):
        reduce_and_store(i, cps[i])


def _sc_logits(idx, table_flat):
    n_b = idx.shape[0]
    b_per_w = n_b // (NC * NS)
    mesh = plsc.VectorSubcoreMesh(core_axis_name="c", subcore_axis_name="s")
    f = functools.partial(
        pl.kernel,
        mesh=mesh,
        out_type=jax.ShapeDtypeStruct((n_b, CPAD), jnp.float32),
        scratch_types=[
            pltpu.VMEM((b_per_w, 2, 128), jnp.int32),
            pltpu.VMEM((b_per_w, 2 * NUM_CLASSES * 128), jnp.int32),
            pltpu.VMEM((b_per_w, 2 * NUM_CLASSES * 128), jnp.float32),
            pltpu.VMEM((CPAD,), jnp.float32),
            pltpu.SemaphoreType.DMA,
        ],
        compiler_params=pltpu.CompilerParams(use_tc_tiling_on_sc=False,
                                             needs_layout_passes=False),
    )(_sc_body)
    return f(idx, table_flat)


def kernel(inputs, centers, class_log_probs):
    b, h, w = inputs.shape
    p = PATCH
    gh, gw = h // p, w // p
    pp = gh * gw
    patches = (inputs.reshape(b, gh, p, gw, p)
               .transpose(0, 1, 3, 2, 4)
               .reshape(b * pp, p * p))
    idx = _tc_tokens(patches, centers)

    idx = jnp.pad(idx.reshape(b, pp), ((0, 0), (0, 256 - pp))).reshape(b, 2, 128)

    out = _sc_logits(idx, _tc_flatten(class_log_probs))
    return out[:, :NUM_CLASSES]


# speedup vs baseline: 1.2327x; 1.2292x over previous
import functools

import jax
import jax.numpy as jnp
from jax import lax
from jax.experimental import pallas as pl
from jax.experimental.pallas import tpu as pltpu
from jax.experimental.pallas import tpu_sc as plsc

PATCH = 16
K = 1024
NUM_CLASSES = 10
CPAD = 16
ROWS_BLK = 896
NC, NS = 2, 16


def _tok_body(x_ref, c_ref, o_ref):
    x = x_ref[...]
    c = c_ref[...]
    xc = lax.dot_general(c, x, (((1,), (1,)), ((), ())),
                         preferred_element_type=jnp.float32)
    c2 = jnp.sum(c * c, axis=1)
    dist = c2[:, None] - 2.0 * xc
    m = jnp.min(dist, axis=0, keepdims=True)
    rows = lax.broadcasted_iota(jnp.int32, dist.shape, 0)
    tok = jnp.min(jnp.where(dist == m, rows, K), axis=0)
    g = pl.program_id(0) * ROWS_BLK + lax.broadcasted_iota(
        jnp.int32, (ROWS_BLK,), 0)
    idx = tok + (g % 196) * K
    o_ref[...] = idx.reshape(1, 1, ROWS_BLK)


def _flat_body(x_ref, o_ref):
    o_ref[...] = x_ref[...].reshape(o_ref.shape)


def _tc_flatten(clp):
    c, p, k = clp.shape
    return pl.pallas_call(
        _flat_body,
        grid=(c,),
        in_specs=[pl.BlockSpec((1, p, k), lambda i: (i, 0, 0))],
        out_specs=pl.BlockSpec((p * k,), lambda i: (i,)),
        out_shape=jax.ShapeDtypeStruct((c * p * k,), jnp.float32),
    )(clp)


def _tc_tokens(patches, centers):
    n = patches.shape[0]
    grid = n // ROWS_BLK
    out = pl.pallas_call(
        _tok_body,
        grid=(grid,),
        in_specs=[
            pl.BlockSpec((ROWS_BLK, patches.shape[1]), lambda i: (i, 0)),
            pl.BlockSpec(centers.shape, lambda i: (0, 0)),
        ],
        out_specs=pl.BlockSpec((1, 1, ROWS_BLK), lambda i: (i, 0, 0)),
        out_shape=jax.ShapeDtypeStruct((grid, 1, ROWS_BLK), jnp.int32),
    )(patches, centers)
    return out.reshape(n)


_PK = 196 * K
_VALID = 196


def _sc_body(idx_hbm, table_hbm, out_hbm, idx_v, idxc_v, rows_v, acc_v, sem):
    cid = lax.axis_index("c")
    sid = lax.axis_index("s")
    wid = sid * NC + cid
    n_b = idx_hbm.shape[0]
    b_per_w = n_b // (NC * NS)
    lane = lax.iota(jnp.int32, CPAD)

    def build_and_fire(i):
        b = wid * b_per_w + i
        pltpu.sync_copy(idx_hbm.at[b], idx_v.at[i])
        for c in range(NUM_CLASSES):
            off = jnp.full((CPAD,), c * _PK, jnp.int32)
            for h in range(2):
                for j in range(8):
                    idxc_v[i, pl.ds((2 * c + h) * 128 + 16 * j, 16)] = (
                        idx_v[i, h, pl.ds(16 * j, 16)] + off)
        return pltpu.async_copy(table_hbm.at[idxc_v.at[i]], rows_v.at[i], sem)

    def reduce_and_store(i, cp):
        b = wid * b_per_w + i
        cp.wait()
        out_acc = jnp.zeros((CPAD,), jnp.float32)
        for c in range(NUM_CLASSES):
            a0 = jnp.zeros((CPAD,), jnp.float32)
            a1 = jnp.zeros((CPAD,), jnp.float32)
            for j in range(8):
                a0 = a0 + rows_v[i, pl.ds(2 * c * 128 + 16 * j, 16)]
            for j in range(4):
                a1 = a1 + rows_v[i, pl.ds((2 * c + 1) * 128 + 16 * j, 16)]
            tail = rows_v[i, pl.ds((2 * c + 1) * 128 + 64, 16)]
            a1 = a1 + jnp.where(lane < _VALID - 192, tail, 0.0)
            s = jnp.sum(a0 + a1)
            out_acc = out_acc + jnp.where(lane == c, s, 0.0)
        acc_v[...] = out_acc
        pltpu.sync_copy(acc_v, out_hbm.at[b])

    cps = [build_and_fire(i) for i in range(b_per_w)]
    for i in range(b_per_w):
        reduce_and_store(i, cps[i])


def _sc_logits(idx, table_flat):
    n_b = idx.shape[0]
    b_per_w = n_b // (NC * NS)
    mesh = plsc.VectorSubcoreMesh(core_axis_name="c", subcore_axis_name="s")
    f = functools.partial(
        pl.kernel,
        mesh=mesh,
        out_type=jax.ShapeDtypeStruct((n_b, CPAD), jnp.float32),
        scratch_types=[
            pltpu.VMEM((b_per_w, 2, 128), jnp.int32),
            pltpu.VMEM((b_per_w, 2 * NUM_CLASSES * 128), jnp.int32),
            pltpu.VMEM((b_per_w, 2 * NUM_CLASSES * 128), jnp.float32),
            pltpu.VMEM((CPAD,), jnp.float32),
            pltpu.SemaphoreType.DMA,
        ],
        compiler_params=pltpu.CompilerParams(use_tc_tiling_on_sc=False,
                                             needs_layout_passes=False),
    )(_sc_body)
    return f(idx, table_flat)


def kernel(inputs, centers, class_log_probs):
    b, h, w = inputs.shape
    p = PATCH
    gh, gw = h // p, w // p
    pp = gh * gw
    patches = (inputs.reshape(b, gh, p, gw, p)
               .transpose(0, 1, 3, 2, 4)
               .reshape(b * pp, p * p))
    idx = _tc_tokens(patches, centers)

    idx = jnp.pad(idx.reshape(b, pp), ((0, 0), (0, 256 - pp))).reshape(b, 2, 128)

    out = _sc_logits(idx, _tc_flatten(class_log_probs))
    return out[:, :NUM_CLASSES]
